# Initial kernel scaffold; baseline (speedup 1.0000x reference)
import jax
import jax.numpy as jnp
from jax.experimental import pallas as pl


def kernel(length_q, length_k, embeddings_table):
    raise NotImplementedError("write your pallas kernel here")


# speedup vs baseline: 6.3894x; 6.3894x over previous
import jax
import jax.numpy as jnp
from jax import lax
from jax.experimental import pallas as pl
from jax.experimental.pallas import tpu as pltpu
from jax.experimental.pallas import tpu_sc as plsc

HEAD_DIM = 64
MAX_REL = 128
LEN_Q = 2048
LEN_K = 2048
TAB_ROWS = 2 * MAX_REL + 1
NC, NS = 2, 16
NW = NC * NS
ROWS_PER_W = LEN_Q // NW
E_ROWS = LEN_Q + LEN_K
FILL = LEN_K - MAX_REL - 1
ROW_ELEMS = LEN_K * HEAD_DIM
CHUNK = 64
N_CHUNK = 30
JB = 2
CB = LEN_K // JB
WIN = CB + ROWS_PER_W
E_ELEMS = E_ROWS * HEAD_DIM


def _body(table_hbm, out_hbm, e_hbm, win_v, a_v, t_v, row_v, sem):
    c = lax.axis_index("c")
    s = lax.axis_index("s")
    e_base = c * E_ELEMS

    @pl.when(s < 2)
    def _build_fill():
        src = pl.multiple_of(s * ((TAB_ROWS - 1) * HEAD_DIM), HEAD_DIM)
        pltpu.sync_copy(table_hbm.at[pl.ds(src, HEAD_DIM)], row_v)
        v0 = row_v[pl.ds(0, 16)]
        v1 = row_v[pl.ds(16, 16)]
        v2 = row_v[pl.ds(32, 16)]
        v3 = row_v[pl.ds(48, 16)]
        for r in range(CHUNK):
            a_v[pl.ds(r * HEAD_DIM, 16)] = v0
            a_v[pl.ds(r * HEAD_DIM + 16, 16)] = v1
            a_v[pl.ds(r * HEAD_DIM + 32, 16)] = v2
            a_v[pl.ds(r * HEAD_DIM + 48, 16)] = v3
        dst0 = e_base + s * ((FILL + TAB_ROWS) * HEAD_DIM)
        for k in range(N_CHUNK):
            dst = pl.multiple_of(dst0 + k * CHUNK * HEAD_DIM, HEAD_DIM)
            pltpu.sync_copy(a_v, e_hbm.at[pl.ds(dst, CHUNK * HEAD_DIM)])

    @pl.when(s == 2)
    def _copy_table():
        pltpu.sync_copy(table_hbm, t_v)
        dst = pl.multiple_of(e_base + FILL * HEAD_DIM, HEAD_DIM)
        pltpu.sync_copy(t_v, e_hbm.at[pl.ds(dst, TAB_ROWS * HEAD_DIM)])

    plsc.subcore_barrier()

    wid = s * NC + c
    base = wid * ROWS_PER_W

    for jb in range(JB):
        w0 = pl.multiple_of(
            e_base + (jb * CB + LEN_Q - 1 - base - (ROWS_PER_W - 1)) * HEAD_DIM,
            HEAD_DIM)
        pltpu.sync_copy(e_hbm.at[pl.ds(w0, WIN * HEAD_DIM)], win_v)

        def fire(t, carry):
            i = base + t
            src = pl.multiple_of((ROWS_PER_W - 1 - t) * HEAD_DIM, HEAD_DIM)
            dst = pl.multiple_of(i * ROW_ELEMS + jb * CB * HEAD_DIM,
                                 CB * HEAD_DIM)
            pltpu.async_copy(win_v.at[pl.ds(src, CB * HEAD_DIM)],
                             out_hbm.at[pl.ds(dst, CB * HEAD_DIM)], sem)
            return carry

        lax.fori_loop(0, ROWS_PER_W, fire, 0)

        def drain(t, carry):
            pltpu.make_async_copy(e_hbm.at[pl.ds(0, CB * HEAD_DIM)],
                                  win_v.at[pl.ds(0, CB * HEAD_DIM)],
                                  sem).wait()
            return carry

        lax.fori_loop(0, ROWS_PER_W, drain, 0)


def kernel(length_q, length_k, embeddings_table):
    del length_q, length_k
    mesh = plsc.VectorSubcoreMesh(
        core_axis_name="c", subcore_axis_name="s",
        num_cores=NC, num_subcores=NS)
    flat, _ = pl.kernel(
        _body,
        out_type=(
            jax.ShapeDtypeStruct((LEN_Q * LEN_K * HEAD_DIM,), jnp.float32),
            jax.ShapeDtypeStruct((NC * E_ELEMS,), jnp.float32),
        ),
        mesh=mesh,
        scratch_types=[
            pltpu.VMEM((WIN * HEAD_DIM,), jnp.float32),
            pltpu.VMEM((CHUNK * HEAD_DIM,), jnp.float32),
            pltpu.VMEM((TAB_ROWS * HEAD_DIM,), jnp.float32),
            pltpu.VMEM((HEAD_DIM,), jnp.float32),
            pltpu.SemaphoreType.DMA,
        ],
    )(embeddings_table.reshape(-1))
    return flat.reshape(LEN_Q, LEN_K, HEAD_DIM)


# speedup vs baseline: 8.0332x; 1.2573x over previous
import jax
import jax.numpy as jnp
from jax import lax
from jax.experimental import pallas as pl
from jax.experimental.pallas import tpu as pltpu
from jax.experimental.pallas import tpu_sc as plsc

HEAD_DIM = 64
MAX_REL = 128
LEN_Q = 2048
LEN_K = 2048
TAB_ROWS = 2 * MAX_REL + 1
NC, NS = 2, 16
NW = NC * NS
ROWS_PER_W = LEN_Q // NW
E_COLS = LEN_Q + LEN_K
FILL = LEN_K - MAX_REL - 1
ROW_ELEMS = LEN_K * HEAD_DIM
DT = HEAD_DIM // 8
PB_ELEMS = (LEN_K // 128) * 8 * 128
D_PER_W = HEAD_DIM // NS
TAB_P = 264


def _body(tt_hbm, e0_hbm, e2_hbm, out_hbm, e_hbm,
          slab_v, pb_v, tt_v, ed0_v, ed2_v, et_v, sem, sem2):
    c = lax.axis_index("c")
    s = lax.axis_index("s")

    src_t = pl.multiple_of(s * (D_PER_W * TAB_P), 8)
    pltpu.sync_copy(tt_hbm.at[pl.ds(src_t, D_PER_W * TAB_P)], tt_v)
    src_e = pl.multiple_of(s * (D_PER_W * 16), 8)
    pltpu.sync_copy(e0_hbm.at[pl.ds(src_e, D_PER_W * 16)], ed0_v)
    pltpu.sync_copy(e2_hbm.at[pl.ds(src_e, D_PER_W * 16)], ed2_v)
    for q in range(D_PER_W):
        d = s * D_PER_W + q
        f0 = ed0_v[pl.ds(q * 16, 16)]
        f256 = ed2_v[pl.ds(q * 16, 16)]
        for g in range(120):
            et_v[pl.ds(16 * g, 16)] = f0
            et_v[pl.ds(FILL + TAB_ROWS + 16 * g, 16)] = f256
        for g in range(16):
            mid = tt_v[pl.ds(q * TAB_P + 16 * g, 16)]
            et_v[pl.ds(FILL + 16 * g, 16)] = mid
        last = tt_v[pl.ds(q * TAB_P + TAB_ROWS - 16, 16)]
        et_v[pl.ds(FILL + TAB_ROWS - 16, 16)] = last
        dst = pl.multiple_of((c * HEAD_DIM + d) * E_COLS, 8)
        pltpu.sync_copy(et_v, e_hbm.at[pl.ds(dst, E_COLS)])

    plsc.subcore_barrier()

    wid = s * NC + c
    base = wid * ROWS_PER_W

    for db in range(DT):
        def fetch(r, carry):
            src = pl.multiple_of((c * HEAD_DIM + db * 8 + r) * E_COLS, 8)
            pltpu.async_copy(e_hbm.at[pl.ds(src, E_COLS)],
                             slab_v.at[pl.ds(r * E_COLS, E_COLS)], sem2)
            return carry

        lax.fori_loop(0, 8, fetch, 0)

        def fetch_drain(r, carry):
            pltpu.make_async_copy(e_hbm.at[pl.ds(0, E_COLS)],
                                  slab_v.at[pl.ds(0, E_COLS)], sem2).wait()
            return carry

        lax.fori_loop(0, 8, fetch_drain, 0)

        def row_body(t, carry):
            i = base + t
            off = LEN_Q - 1 - i
            par = (t % 2) * PB_ELEMS

            @pl.when(t >= 2)
            def _wait_prev():
                pltpu.make_async_copy(out_hbm.at[pl.ds(0, PB_ELEMS)],
                                      pb_v.at[pl.ds(0, PB_ELEMS)],
                                      sem).wait()

            def jt_body(jt, carry2):
                s0 = off + jt * 128
                p0 = par + jt * 1024
                for dr in range(8):
                    for l in range(8):
                        v = slab_v[pl.ds(dr * E_COLS + s0 + l * 16, 16)]
                        pb_v[pl.ds(p0 + dr * 128 + l * 16, 16)] = v
                return carry2

            lax.fori_loop(0, LEN_K // 128, jt_body, 0)
            dst = pl.multiple_of(i * ROW_ELEMS + db * PB_ELEMS, 8)
            pltpu.async_copy(pb_v.at[pl.ds(par, PB_ELEMS)],
                             out_hbm.at[pl.ds(dst, PB_ELEMS)], sem)
            return carry

        lax.fori_loop(0, ROWS_PER_W, row_body, 0)

        def tail_drain(r, carry):
            pltpu.make_async_copy(out_hbm.at[pl.ds(0, PB_ELEMS)],
                                  pb_v.at[pl.ds(0, PB_ELEMS)], sem).wait()
            return carry

        lax.fori_loop(0, 2, tail_drain, 0)


def kernel(length_q, length_k, embeddings_table):
    del length_q, length_k
    mesh = plsc.VectorSubcoreMesh(
        core_axis_name="c", subcore_axis_name="s",
        num_cores=NC, num_subcores=NS)
    table_t = jnp.pad(embeddings_table.T, ((0, 0), (0, TAB_P - TAB_ROWS)))
    edge0 = jnp.broadcast_to(
        embeddings_table[0][:, None], (HEAD_DIM, 16))
    edge2 = jnp.broadcast_to(
        embeddings_table[TAB_ROWS - 1][:, None], (HEAD_DIM, 16))
    flat, _ = pl.kernel(
        _body,
        out_type=(
            jax.ShapeDtypeStruct((LEN_Q * LEN_K * HEAD_DIM,), jnp.float32),
            jax.ShapeDtypeStruct((NC * HEAD_DIM * E_COLS,), jnp.float32),
        ),
        mesh=mesh,
        scratch_types=[
            pltpu.VMEM((8 * E_COLS,), jnp.float32),
            pltpu.VMEM((2 * PB_ELEMS,), jnp.float32),
            pltpu.VMEM((D_PER_W * TAB_P,), jnp.float32),
            pltpu.VMEM((D_PER_W * 16,), jnp.float32),
            pltpu.VMEM((D_PER_W * 16,), jnp.float32),
            pltpu.VMEM((E_COLS,), jnp.float32),
            pltpu.SemaphoreType.DMA,
            pltpu.SemaphoreType.DMA,
        ],
    )(table_t.reshape(-1), edge0.reshape(-1), edge2.reshape(-1))
    out5 = flat.reshape(LEN_Q, DT, LEN_K // 128, 8, 128)
    return out5.transpose(0, 2, 4, 1, 3).reshape(LEN_Q, LEN_K, HEAD_DIM)


# speedup vs baseline: 30.5229x; 3.7996x over previous
import jax
import jax.numpy as jnp
from jax import lax
from jax.experimental import pallas as pl
from jax.experimental.pallas import tpu as pltpu
from jax.experimental.pallas import tpu_sc as plsc

HEAD_DIM = 64
MAX_REL = 128
LEN_Q = 2048
LEN_K = 2048
TAB_ROWS = 2 * MAX_REL + 1
NC, NS = 2, 16
NW = NC * NS
ROWS_PER_W = LEN_Q // NW
E_COLS = LEN_Q + LEN_K
FILL = LEN_K - MAX_REL - 1
DT = HEAD_DIM // 8
NJT = LEN_K // 128
D_PER_W = HEAD_DIM // NS
TAB_P = 264
RG = 16
NG = ROWS_PER_W // RG
SLABW = 576
MIDT = 4


def _body(tt_hbm, e0_hbm, e2_hbm, out_hbm, e_hbm,
          slab_v, pb_v, c0_v, c2_v, tt_v, ed0_v, ed2_v, et_v,
          sem_c, sem_m, sem_f):
    c = lax.axis_index("c")
    s = lax.axis_index("s")

    src_t = pl.multiple_of(s * (D_PER_W * TAB_P), 8)
    pltpu.sync_copy(tt_hbm.at[pl.ds(src_t, D_PER_W * TAB_P)], tt_v)
    pltpu.sync_copy(e0_hbm, ed0_v)
    pltpu.sync_copy(e2_hbm, ed2_v)
    for q in range(D_PER_W):
        d = s * D_PER_W + q
        f0 = ed0_v[pl.ds(d * 16, 16)]
        f256 = ed2_v[pl.ds(d * 16, 16)]
        for g in range(120):
            et_v[pl.ds(16 * g, 16)] = f0
            et_v[pl.ds(FILL + TAB_ROWS + 16 * g, 16)] = f256
        for g in range(16):
            mid = tt_v[pl.ds(q * TAB_P + 16 * g, 16)]
            et_v[pl.ds(FILL + 16 * g, 16)] = mid
        last = tt_v[pl.ds(q * TAB_P + TAB_ROWS - 16, 16)]
        et_v[pl.ds(FILL + TAB_ROWS - 16, 16)] = last
        dst = pl.multiple_of((c * HEAD_DIM + d) * E_COLS, 8)
        pltpu.sync_copy(et_v, e_hbm.at[pl.ds(dst, E_COLS)])

    plsc.subcore_barrier()


    wid = s * NC + c
    base = wid * ROWS_PER_W

    def db_body(db, carry):
        vs0 = [ed0_v[pl.ds((8 * db + dr) * 16, 16)] for dr in range(8)]
        vs2 = [ed2_v[pl.ds((8 * db + dr) * 16, 16)] for dr in range(8)]

        def img_body(u, carry2):
            for dr in range(8):
                for l in range(8):
                    c0_v[u, dr, pl.ds(16 * l, 16)] = vs0[dr]
                    c2_v[u, dr, pl.ds(16 * l, 16)] = vs2[dr]
            return carry2

        lax.fori_loop(0, RG, img_body, 0)

        for g in range(NG):
            rb = base + RG * g
            mid_lo = jnp.clip(
                lax.shift_right_arithmetic(rb - MAX_REL, 7), 0, NJT - MIDT)
            w0 = pl.multiple_of(
                128 * mid_lo + (LEN_Q - 1 - (RG - 1)) - rb, 8)

            def f_body(r, carry2):
                src = pl.multiple_of(
                    (c * HEAD_DIM + 8 * db + r) * E_COLS + w0, 8)
                pltpu.async_copy(e_hbm.at[pl.ds(src, SLABW)],
                                 slab_v.at[pl.ds(r * SLABW, SLABW)], sem_f)
                return carry2

            lax.fori_loop(0, 8, f_body, 0)

            def f_drain(r, carry2):
                pltpu.make_async_copy(e_hbm.at[pl.ds(0, SLABW)],
                                      slab_v.at[pl.ds(0, SLABW)],
                                      sem_f).wait()
                return carry2

            lax.fori_loop(0, 8, f_drain, 0)

            def c0_body(jt, carry2):
                pltpu.async_copy(c0_v, out_hbm.at[pl.ds(rb, RG), db, jt],
                                 sem_c)
                return carry2

            lax.fori_loop(0, mid_lo, c0_body, 0)

            def c2_body(jt, carry2):
                pltpu.async_copy(c2_v, out_hbm.at[pl.ds(rb, RG), db, jt],
                                 sem_c)
                return carry2

            lax.fori_loop(mid_lo + MIDT, NJT, c2_body, 0)

            for m in range(MIDT):
                k = g * MIDT + m
                par = k % 2
                if k >= 2:
                    pltpu.make_async_copy(
                        out_hbm.at[pl.ds(0, RG), 0, 0],
                        pb_v.at[par], sem_m).wait()

                def a_body(u, carry2):
                    s0 = 128 * m + (RG - 1) - u
                    for dr in range(8):
                        for l in range(8):
                            v = slab_v[pl.ds(dr * SLABW + s0 + 16 * l, 16)]
                            pb_v[par, u, dr, pl.ds(16 * l, 16)] = v
                    return carry2

                lax.fori_loop(0, RG, a_body, 0)
                pltpu.async_copy(
                    pb_v.at[par], out_hbm.at[pl.ds(rb, RG), db, mid_lo + m],
                    sem_m)

        def c_drain(r, carry2):
            pltpu.make_async_copy(out_hbm.at[pl.ds(0, RG), 0, 0],
                                  c0_v, sem_c).wait()
            return carry2

        lax.fori_loop(0, NG * (NJT - MIDT), c_drain, 0)
        for _ in range(2):
            pltpu.make_async_copy(out_hbm.at[pl.ds(0, RG), 0, 0],
                                  pb_v.at[0], sem_m).wait()
        return carry

    lax.fori_loop(0, DT, db_body, 0)


def kernel(length_q, length_k, embeddings_table):
    del length_q, length_k
    mesh = plsc.VectorSubcoreMesh(
        core_axis_name="c", subcore_axis_name="s",
        num_cores=NC, num_subcores=NS)
    table_t = jnp.pad(embeddings_table.T, ((0, 0), (0, TAB_P - TAB_ROWS)))
    edge0 = jnp.broadcast_to(
        embeddings_table[0][:, None], (HEAD_DIM, 16))
    edge2 = jnp.broadcast_to(
        embeddings_table[TAB_ROWS - 1][:, None], (HEAD_DIM, 16))
    out5, _ = pl.kernel(
        _body,
        out_type=(
            jax.ShapeDtypeStruct((LEN_Q, DT, NJT, 8, 128), jnp.float32),
            jax.ShapeDtypeStruct((NC * HEAD_DIM * E_COLS,), jnp.float32),
        ),
        mesh=mesh,
        scratch_types=[
            pltpu.VMEM((8 * SLABW,), jnp.float32),
            pltpu.VMEM((2, RG, 8, 128), jnp.float32),
            pltpu.VMEM((RG, 8, 128), jnp.float32),
            pltpu.VMEM((RG, 8, 128), jnp.float32),
            pltpu.VMEM((D_PER_W * TAB_P,), jnp.float32),
            pltpu.VMEM((HEAD_DIM * 16,), jnp.float32),
            pltpu.VMEM((HEAD_DIM * 16,), jnp.float32),
            pltpu.VMEM((E_COLS,), jnp.float32),
            pltpu.SemaphoreType.DMA,
            pltpu.SemaphoreType.DMA,
            pltpu.SemaphoreType.DMA,
        ],
    )(table_t.reshape(-1), edge0.reshape(-1), edge2.reshape(-1))
    return out5.transpose(0, 2, 4, 1, 3).reshape(LEN_Q, LEN_K, HEAD_DIM)


# speedup vs baseline: 38.3943x; 1.2579x over previous
import jax
import jax.numpy as jnp
from jax import lax
from jax.experimental import pallas as pl
from jax.experimental.pallas import tpu as pltpu
from jax.experimental.pallas import tpu_sc as plsc

HEAD_DIM = 64
MAX_REL = 128
LEN_Q = 2048
LEN_K = 2048
TAB_ROWS = 2 * MAX_REL + 1
NC, NS = 2, 16
NW = NC * NS
ROWS_PER_W = LEN_Q // NW
E_COLS = LEN_Q + LEN_K
FILL = LEN_K - MAX_REL - 1
DT = HEAD_DIM // 8
NJT = LEN_K // 128
D_PER_W = HEAD_DIM // NS
TAB_P = 264
RG = 16
NG = ROWS_PER_W // RG
SLABW = 576
MIDT = 4


def _body(tt_hbm, e0_hbm, e2_hbm, out_hbm, e_hbm,
          slab_v, pb_v, c0_v, c2_v, tt_v, ed0_v, ed2_v, et_v,
          sem_c, sem_m, sem_f):
    c = lax.axis_index("c")
    s = lax.axis_index("s")

    src_t = pl.multiple_of(s * (D_PER_W * TAB_P), 8)
    pltpu.sync_copy(tt_hbm.at[pl.ds(src_t, D_PER_W * TAB_P)], tt_v)
    pltpu.sync_copy(e0_hbm, ed0_v)
    pltpu.sync_copy(e2_hbm, ed2_v)
    for q in range(D_PER_W):
        d = s * D_PER_W + q
        f0 = ed0_v[pl.ds(d * 16, 16)]
        f256 = ed2_v[pl.ds(d * 16, 16)]
        for g in range(120):
            et_v[pl.ds(16 * g, 16)] = f0
            et_v[pl.ds(FILL + TAB_ROWS + 16 * g, 16)] = f256
        for g in range(16):
            mid = tt_v[pl.ds(q * TAB_P + 16 * g, 16)]
            et_v[pl.ds(FILL + 16 * g, 16)] = mid
        last = tt_v[pl.ds(q * TAB_P + TAB_ROWS - 16, 16)]
        et_v[pl.ds(FILL + TAB_ROWS - 16, 16)] = last
        dst = pl.multiple_of((c * HEAD_DIM + d) * E_COLS, 8)
        pltpu.sync_copy(et_v, e_hbm.at[pl.ds(dst, E_COLS)])

    plsc.subcore_barrier()


    wid = s * NC + c
    base = wid * ROWS_PER_W

    def db_body(db, carry):
        vs0 = [ed0_v[pl.ds((8 * db + dr) * 16, 16)] for dr in range(8)]
        vs2 = [ed2_v[pl.ds((8 * db + dr) * 16, 16)] for dr in range(8)]

        def img_body(u, carry2):
            for dr in range(8):
                for l in range(8):
                    c0_v[u, dr, pl.ds(16 * l, 16)] = vs0[dr]
                    c2_v[u, dr, pl.ds(16 * l, 16)] = vs2[dr]
            return carry2

        lax.fori_loop(0, RG, img_body, 0)

        for g in range(NG):
            rb = base + RG * g
            mid_lo = jnp.clip(
                lax.shift_right_arithmetic(rb - MAX_REL, 7), 0, NJT - MIDT)
            w0 = pl.multiple_of(
                128 * mid_lo + (LEN_Q - 1 - (RG - 1)) - rb, 8)

            def f_body(r, carry2):
                src = pl.multiple_of(
                    (c * HEAD_DIM + 8 * db + r) * E_COLS + w0, 8)
                pltpu.async_copy(e_hbm.at[pl.ds(src, SLABW)],
                                 slab_v.at[pl.ds(r * SLABW, SLABW)], sem_f)
                return carry2

            lax.fori_loop(0, 8, f_body, 0)

            def f_drain(r, carry2):
                pltpu.make_async_copy(e_hbm.at[pl.ds(0, SLABW)],
                                      slab_v.at[pl.ds(0, SLABW)],
                                      sem_f).wait()
                return carry2

            lax.fori_loop(0, 8, f_drain, 0)

            def c0_body(jt, carry2):
                pltpu.async_copy(c0_v, out_hbm.at[pl.ds(rb, RG), db, jt],
                                 sem_c)
                return carry2

            lax.fori_loop(0, mid_lo, c0_body, 0)

            def c2_body(jt, carry2):
                pltpu.async_copy(c2_v, out_hbm.at[pl.ds(rb, RG), db, jt],
                                 sem_c)
                return carry2

            lax.fori_loop(mid_lo + MIDT, NJT, c2_body, 0)

            for m in range(MIDT):
                k = g * MIDT + m
                par = k % 2
                if k >= 2:
                    pltpu.make_async_copy(
                        out_hbm.at[pl.ds(0, RG), 0, 0],
                        pb_v.at[par], sem_m).wait()

                def a_body(u, carry2):
                    s0 = 128 * m + (RG - 1) - u
                    for drp in range(4):
                        vv = [
                            slab_v[pl.ds((2 * drp + h) * SLABW
                                         + s0 + 16 * l, 16)]
                            for h in range(2) for l in range(8)
                        ]
                        for h in range(2):
                            for l in range(8):
                                pb_v[par, u, 2 * drp + h,
                                     pl.ds(16 * l, 16)] = vv[8 * h + l]
                    return carry2

                lax.fori_loop(0, RG, a_body, 0)
                pltpu.async_copy(
                    pb_v.at[par], out_hbm.at[pl.ds(rb, RG), db, mid_lo + m],
                    sem_m)

        def c_drain(r, carry2):
            pltpu.make_async_copy(out_hbm.at[pl.ds(0, RG), 0, 0],
                                  c0_v, sem_c).wait()
            return carry2

        lax.fori_loop(0, NG * (NJT - MIDT), c_drain, 0)
        for _ in range(2):
            pltpu.make_async_copy(out_hbm.at[pl.ds(0, RG), 0, 0],
                                  pb_v.at[0], sem_m).wait()
        return carry

    lax.fori_loop(0, DT, db_body, 0)


def kernel(length_q, length_k, embeddings_table):
    del length_q, length_k
    mesh = plsc.VectorSubcoreMesh(
        core_axis_name="c", subcore_axis_name="s",
        num_cores=NC, num_subcores=NS)
    table_t = jnp.pad(embeddings_table.T, ((0, 0), (0, TAB_P - TAB_ROWS)))
    edge0 = jnp.broadcast_to(
        embeddings_table[0][:, None], (HEAD_DIM, 16))
    edge2 = jnp.broadcast_to(
        embeddings_table[TAB_ROWS - 1][:, None], (HEAD_DIM, 16))
    out5, _ = pl.kernel(
        _body,
        out_type=(
            jax.ShapeDtypeStruct((LEN_Q, DT, NJT, 8, 128), jnp.float32),
            jax.ShapeDtypeStruct((NC * HEAD_DIM * E_COLS,), jnp.float32),
        ),
        mesh=mesh,
        scratch_types=[
            pltpu.VMEM((8 * SLABW,), jnp.float32),
            pltpu.VMEM((2, RG, 8, 128), jnp.float32),
            pltpu.VMEM((RG, 8, 128), jnp.float32),
            pltpu.VMEM((RG, 8, 128), jnp.float32),
            pltpu.VMEM((D_PER_W * TAB_P,), jnp.float32),
            pltpu.VMEM((HEAD_DIM * 16,), jnp.float32),
            pltpu.VMEM((HEAD_DIM * 16,), jnp.float32),
            pltpu.VMEM((E_COLS,), jnp.float32),
            pltpu.SemaphoreType.DMA,
            pltpu.SemaphoreType.DMA,
            pltpu.SemaphoreType.DMA,
        ],
    )(table_t.reshape(-1), edge0.reshape(-1), edge2.reshape(-1))
    return out5.transpose(0, 2, 4, 1, 3).reshape(LEN_Q, LEN_K, HEAD_DIM)
